# baseline (speedup 1.0000x reference)
import functools

import jax
import jax.numpy as jnp
from jax import lax
from jax.experimental import pallas as pl
from jax.experimental.pallas import tpu as pltpu
from jax.experimental.pallas import tpu_sc as plsc

VOCAB = 100000
D = 128
BATCH = 4096
SEQ = 200
N = BATCH * SEQ
EPS = 1e-12

NC = 2
NS = 16
NW = NC * NS
C = 64
NBUF = 4

S = 4
N_S = N // S
_LN_ROWS = 8192


def _make_sc_gather(n_rows):
    per_w = n_rows // NW
    nchunk = per_w // C
    assert per_w % C == 0 and nchunk >= 4

    def body(ids_hbm, table_hbm, out_hbm, idx_all, bufs, ibufs, sems_in,
             sems_out):
        wid = lax.axis_index("s") * NC + lax.axis_index("c")
        base = wid * per_w

        pltpu.sync_copy(ids_hbm.at[pl.ds(base, per_w)], idx_all)

        def fire_gather(g, s):
            pltpu.async_copy(
                table_hbm.at[idx_all.at[pl.ds(g * C, C)]], bufs[s], sems_in[s])

        def wait_gather(g, s):
            pltpu.make_async_copy(
                table_hbm.at[idx_all.at[pl.ds(g * C, C)]], bufs[s],
                sems_in[s]).wait()

        def fire_write(g, s):
            pltpu.async_copy(
                ibufs[s], out_hbm.at[pl.ds(base + g * C, C)], sems_out[s])

        def wait_write(g, s):
            pltpu.make_async_copy(
                ibufs[s], out_hbm.at[pl.ds(base + g * C, C)],
                sems_out[s]).wait()

        def to_bf16(s):
            evens = lax.iota(jnp.int32, 16) * 2
            cols = [(evens + k * 32, evens + (k * 32 + 1))
                    for k in range(D // 32)]

            @pl.loop(0, C, unroll=2)
            def _row(j):
                row = jnp.full((16,), j, jnp.int32)
                for k, (ce, co) in enumerate(cols):
                    a = plsc.load_gather(bufs[s], [row, ce])
                    b = plsc.load_gather(bufs[s], [row, co])
                    p = plsc.pack(a, b, format=plsc.PackFormat.INTERLEAVED)
                    ibufs[s][j, pl.ds(k * 16, 16)] = plsc.bitcast(p, jnp.int32)

        fire_gather(0, 0)
        fire_gather(1, 1)
        for g in (0, 1):
            s = g % NBUF
            wait_gather(g, s)
            to_bf16(s)
            fire_write(g, s)
            fire_gather(g + 2, (g + 2) % NBUF)

        main_n = ((nchunk - 4) // NBUF) * NBUF

        @pl.loop(2, 2 + main_n, step=NBUF)
        def _outer(g0):
            for b in range(NBUF):
                g = g0 + b
                s = (2 + b) % NBUF
                wait_gather(g, s)
                wait_write(g - 2, (s + 2) % NBUF)
                fire_gather(g + 2, (s + 2) % NBUF)
                to_bf16(s)
                fire_write(g, s)

        for g in range(2 + main_n, nchunk):
            s = g % NBUF
            wait_gather(g, s)
            wait_write(g - 2, (g - 2) % NBUF)
            if g + 2 < nchunk:
                fire_gather(g + 2, (g + 2) % NBUF)
            to_bf16(s)
            fire_write(g, s)
        for g in (nchunk - 2, nchunk - 1):
            wait_write(g, g % NBUF)

    return functools.partial(
        pl.kernel,
        out_type=jax.ShapeDtypeStruct((n_rows, D // 2), jnp.int32),
        mesh=plsc.VectorSubcoreMesh(core_axis_name="c", subcore_axis_name="s"),
        compiler_params=pltpu.CompilerParams(needs_layout_passes=False),
        scratch_types=[
            pltpu.VMEM((per_w,), jnp.int32),
            [pltpu.VMEM((C, D), jnp.float32) for _ in range(NBUF)],
            [pltpu.VMEM((C, D // 2), jnp.int32) for _ in range(NBUF)],
            [pltpu.SemaphoreType.DMA for _ in range(NBUF)],
            [pltpu.SemaphoreType.DMA for _ in range(NBUF)],
        ],
    )(body)


_sc_gather_slice = _make_sc_gather(N_S)


def _ln_math(x_ref, g_ref, b_ref, o_ref):
    x = x_ref[...].astype(jnp.float32)
    mean = jnp.mean(x, axis=1, keepdims=True)
    cent = x - mean
    var = jnp.mean(cent * cent, axis=1, keepdims=True)
    o_ref[...] = cent * lax.rsqrt(var + EPS) * g_ref[...] + b_ref[...]


def _ln_body(x_ref, g_ref, b_ref, o_ref):
    _ln_math(x_ref, g_ref, b_ref, o_ref)


def _ln_body_acc(x_ref, g_ref, b_ref, acc_ref, o_ref):
    del acc_ref
    _ln_math(x_ref, g_ref, b_ref, o_ref)


def _tc_layernorm_slice(i, rows, gamma, beta, acc):
    blocks = N_S // _LN_ROWS
    x_spec = pl.BlockSpec((_LN_ROWS, D), lambda j: (j, 0))
    gb_spec = pl.BlockSpec((1, D), lambda j: (0, 0))
    out_spec = pl.BlockSpec(
        (_LN_ROWS, D), lambda j, i=i: (i * blocks + j, 0))
    out_shape = jax.ShapeDtypeStruct((N, D), jnp.float32)
    g2, b2 = gamma.reshape(1, D), beta.reshape(1, D)
    if acc is None:
        return pl.pallas_call(
            _ln_body,
            grid=(blocks,),
            in_specs=[x_spec, gb_spec, gb_spec],
            out_specs=out_spec,
            out_shape=out_shape,
        )(rows, g2, b2)
    return pl.pallas_call(
        _ln_body_acc,
        grid=(blocks,),
        in_specs=[x_spec, gb_spec, gb_spec,
                  pl.BlockSpec(memory_space=pl.ANY)],
        out_specs=out_spec,
        out_shape=out_shape,
        input_output_aliases={3: 0},
    )(rows, g2, b2, acc)


def kernel(input_ids, table, gamma, beta):
    ids = input_ids.reshape(-1).astype(jnp.int32)
    acc = None
    for i in range(S):
        words_i = _sc_gather_slice(ids[i * N_S:(i + 1) * N_S], table)
        rows_i = lax.bitcast_convert_type(
            words_i, jnp.bfloat16).reshape(N_S, D)
        acc = _tc_layernorm_slice(i, rows_i, gamma, beta, acc)
    return acc.reshape(BATCH, SEQ, D)


# speedup vs baseline: 4.1753x; 4.1753x over previous
import functools

import jax
import jax.numpy as jnp
from jax import lax
from jax.experimental import pallas as pl
from jax.experimental.pallas import tpu as pltpu
from jax.experimental.pallas import tpu_sc as plsc

VOCAB = 100000
D = 128
BATCH = 4096
SEQ = 200
N = BATCH * SEQ
EPS = 1e-12

NC = 2
NS = 16
NW = NC * NS
C = 128
NBUF = 4

S = 4
N_S = N // S
_LN_ROWS = 8192


def _make_sc_gather(n_rows):
    per_w = n_rows // NW
    nchunk = per_w // C
    assert per_w % C == 0 and nchunk >= 8

    def body(ids_hbm, table_hbm, out_hbm, idx_all, bufs, sems_in, sems_out):
        wid = lax.axis_index("s") * NC + lax.axis_index("c")
        base = wid * per_w

        pltpu.sync_copy(ids_hbm.at[pl.ds(base, per_w)], idx_all)

        def fire_gather(g, s):
            pltpu.async_copy(
                table_hbm.at[idx_all.at[pl.ds(g * C, C)]], bufs[s], sems_in[s])

        def wait_gather(g, s):
            pltpu.make_async_copy(
                table_hbm.at[idx_all.at[pl.ds(g * C, C)]], bufs[s],
                sems_in[s]).wait()

        def fire_write(g, s):
            pltpu.async_copy(
                bufs[s], out_hbm.at[pl.ds(base + g * C, C)], sems_out[s])

        def wait_write(g, s):
            pltpu.make_async_copy(
                bufs[s], out_hbm.at[pl.ds(base + g * C, C)], sems_out[s]).wait()

        fire_gather(0, 0)
        fire_gather(1, 1)
        for g in (0, 1):
            s = g % NBUF
            wait_gather(g, s)
            fire_write(g, s)
            fire_gather(g + 2, (g + 2) % NBUF)

        main_n = ((nchunk - 4) // NBUF) * NBUF

        @pl.loop(2, 2 + main_n, step=NBUF)
        def _outer(g0):
            for b in range(NBUF):
                g = g0 + b
                s = (2 + b) % NBUF
                wait_gather(g, s)
                fire_write(g, s)
                wait_write(g - 2, (s + 2) % NBUF)
                fire_gather(g + 2, (s + 2) % NBUF)

        for g in range(2 + main_n, nchunk):
            s = g % NBUF
            wait_gather(g, s)
            if g + 2 < nchunk:
                wait_write(g - 2, (g - 2) % NBUF)
                fire_gather(g + 2, (g + 2) % NBUF)
            else:
                wait_write(g - 2, (g - 2) % NBUF)
            fire_write(g, s)
        for g in (nchunk - 2, nchunk - 1):
            wait_write(g, g % NBUF)

    return functools.partial(
        pl.kernel,
        out_type=jax.ShapeDtypeStruct((n_rows, D), jnp.float32),
        mesh=plsc.VectorSubcoreMesh(core_axis_name="c", subcore_axis_name="s"),
        scratch_types=[
            pltpu.VMEM((per_w,), jnp.int32),
            [pltpu.VMEM((C, D), jnp.float32) for _ in range(NBUF)],
            [pltpu.SemaphoreType.DMA for _ in range(NBUF)],
            [pltpu.SemaphoreType.DMA for _ in range(NBUF)],
        ],
    )(body)


_sc_gather_slice = _make_sc_gather(N_S)


def _ln_math(x_ref, g_ref, b_ref, o_ref):
    x = x_ref[...]
    mean = jnp.mean(x, axis=1, keepdims=True)
    cent = x - mean
    var = jnp.mean(cent * cent, axis=1, keepdims=True)
    o_ref[...] = cent * lax.rsqrt(var + EPS) * g_ref[...] + b_ref[...]


def _ln_body(x_ref, g_ref, b_ref, o_ref):
    _ln_math(x_ref, g_ref, b_ref, o_ref)


def _ln_body_acc(x_ref, g_ref, b_ref, acc_ref, o_ref):
    del acc_ref
    _ln_math(x_ref, g_ref, b_ref, o_ref)


def _tc_layernorm_slice(i, rows, gamma, beta, acc):
    blocks = N_S // _LN_ROWS
    x_spec = pl.BlockSpec((_LN_ROWS, D), lambda j: (j, 0))
    gb_spec = pl.BlockSpec((1, D), lambda j: (0, 0))
    out_spec = pl.BlockSpec(
        (_LN_ROWS, D), lambda j, i=i: (i * blocks + j, 0))
    out_shape = jax.ShapeDtypeStruct((N, D), jnp.float32)
    g2, b2 = gamma.reshape(1, D), beta.reshape(1, D)
    if acc is None:
        return pl.pallas_call(
            _ln_body,
            grid=(blocks,),
            in_specs=[x_spec, gb_spec, gb_spec],
            out_specs=out_spec,
            out_shape=out_shape,
        )(rows, g2, b2)
    return pl.pallas_call(
        _ln_body_acc,
        grid=(blocks,),
        in_specs=[x_spec, gb_spec, gb_spec,
                  pl.BlockSpec(memory_space=pl.ANY)],
        out_specs=out_spec,
        out_shape=out_shape,
        input_output_aliases={3: 0},
    )(rows, g2, b2, acc)


def kernel(input_ids, table, gamma, beta):
    ids = input_ids.reshape(-1).astype(jnp.int32)
    acc = None
    for i in range(S):
        rows_i = _sc_gather_slice(ids[i * N_S:(i + 1) * N_S], table)
        acc = _tc_layernorm_slice(i, rows_i, gamma, beta, acc)
    return acc.reshape(BATCH, SEQ, D)
